# baseline (speedup 1.0000x reference)
import functools

import jax
import jax.numpy as jnp
from jax import lax
from jax.experimental import pallas as pl
from jax.experimental.pallas import tpu as pltpu
from jax.experimental.pallas import tpu_sc as plsc

DIM = 32
N_EMBED = 8192
N_TOKENS = 16384

N_T = 4096
E_C = 2048


def _argmin_body(x_ref, emb_ref, ind_ref, diff_ref):
    x = x_ref[...]
    x2 = jnp.sum(x * x, axis=1, keepdims=True)
    xb = (2.0 * x).astype(jnp.bfloat16)
    n_chunks_half = (N_EMBED // 2) // E_C
    halves = []
    for h in range(2):
        run_min = jnp.full((N_T, 1), jnp.inf, jnp.float32)
        run_idx = jnp.full((N_T, 1), jnp.inf, jnp.float32)
        for c in range(h * n_chunks_half, (h + 1) * n_chunks_half):
            e_blk = emb_ref[:, c * E_C:(c + 1) * E_C]
            mm = jnp.dot(xb, e_blk, preferred_element_type=jnp.float32)
            e2 = jnp.sum(e_blk * e_blk, axis=0, keepdims=True)
            d = (x2 - mm) + e2
            cmin = jnp.min(d, axis=1, keepdims=True)
            iota = (lax.broadcasted_iota(jnp.int32, (N_T, E_C), 1)
                    + c * E_C).astype(jnp.float32)
            cidx = jnp.min(
                jnp.where(d == cmin, iota, jnp.inf),
                axis=1, keepdims=True)
            better = cmin < run_min
            run_min = jnp.where(better, cmin, run_min)
            run_idx = jnp.where(better, cidx, run_idx)
        halves.append((run_min, run_idx))
    (v0, i0), (v1, i1) = halves
    v0_carry = v0.astype(jnp.bfloat16).astype(jnp.float32)
    pick1 = v1 < v0_carry
    run_idx = jnp.where(pick1, i1, i0)
    run_min = jnp.where(pick1, v1, v0)
    ind_ref[...] = run_idx[:, 0].astype(jnp.int32)

    step = pl.program_id(0)

    @pl.when(step == 0)
    def _():
        diff_ref[0, 0] = 0.0

    diff_ref[0, 0] += jnp.sum(run_min)

    @pl.when(step == pl.num_programs(0) - 1)
    def _():
        diff_ref[0, 0] = diff_ref[0, 0] * (1.0 / (N_TOKENS * DIM))


def _argmin_call(flat_x, embed):
    grid = (N_TOKENS // N_T,)
    return pl.pallas_call(
        _argmin_body,
        grid=grid,
        in_specs=[
            pl.BlockSpec((N_T, DIM), lambda i: (i, 0)),
            pl.BlockSpec((DIM, N_EMBED), lambda i: (0, 0)),
        ],
        out_specs=[
            pl.BlockSpec((N_T,), lambda i: (i,)),
            pl.BlockSpec(memory_space=pltpu.SMEM),
        ],
        out_shape=[
            jax.ShapeDtypeStruct((N_TOKENS,), jnp.int32),
            jax.ShapeDtypeStruct((1, 1), jnp.float32),
        ],
    )(flat_x, embed)


def _gather_call(table, ind):
    info = plsc.get_sparse_core_info()
    nw = info.num_cores * info.num_subcores
    b_per_w = N_TOKENS // nw
    n_chunks = b_per_w // 128
    mesh = plsc.VectorSubcoreMesh(core_axis_name="c", subcore_axis_name="s")

    @functools.partial(
        pl.kernel, mesh=mesh,
        compiler_params=pltpu.CompilerParams(use_tc_tiling_on_sc=False),
        out_type=jax.ShapeDtypeStruct((N_TOKENS, DIM), jnp.float32),
        scratch_types=[
            pltpu.VMEM((b_per_w,), jnp.int32),
            pltpu.VMEM((b_per_w, DIM), jnp.float32),
            pltpu.SemaphoreType.DMA,
        ],
    )
    def k(table_hbm, idx_hbm, out_hbm, idx_v, rows_v, sem):
        wid = lax.axis_index("s") * info.num_cores + lax.axis_index("c")
        base = wid * b_per_w
        pltpu.sync_copy(idx_hbm.at[pl.ds(base, b_per_w)], idx_v)
        copies = [
            pltpu.async_copy(
                table_hbm.at[idx_v.at[pl.ds(g * 128, 128)]],
                rows_v.at[pl.ds(g * 128, 128)],
                sem,
            )
            for g in range(n_chunks)
        ]
        for c in copies:
            c.wait()
        pltpu.sync_copy(rows_v, out_hbm.at[pl.ds(base, b_per_w)])

    return k(table, ind)


def kernel(x, embed):
    flat_x = x.reshape(-1, DIM)
    ind, diff = _argmin_call(flat_x, embed)
    table = embed.T
    quantize = _gather_call(table, ind)
    return (
        quantize.reshape(x.shape),
        diff.reshape(()),
        ind.reshape(x.shape[:-1]),
    )


# speedup vs baseline: 1.3013x; 1.3013x over previous
import functools

import jax
import jax.numpy as jnp
from jax import lax
from jax.experimental import pallas as pl
from jax.experimental.pallas import tpu as pltpu
from jax.experimental.pallas import tpu_sc as plsc

DIM = 32
N_EMBED = 8192
N_TOKENS = 16384

N_T = 1024
E_C = 2048


def _argmin_body(x_ref, emb_ref, ind_ref, diff_ref):
    x = x_ref[...]
    x2 = jnp.sum(x * x, axis=1, keepdims=True)
    xb = (2.0 * x).astype(jnp.bfloat16)
    n_chunks_half = (N_EMBED // 2) // E_C
    halves = []
    for h in range(2):
        run_min = jnp.full((N_T, 1), jnp.inf, jnp.float32)
        run_idx = jnp.full((N_T, 1), jnp.inf, jnp.float32)
        for c in range(h * n_chunks_half, (h + 1) * n_chunks_half):
            e_blk = emb_ref[:, c * E_C:(c + 1) * E_C]
            mm = jnp.dot(xb, e_blk, preferred_element_type=jnp.float32)
            e2 = jnp.sum(e_blk * e_blk, axis=0, keepdims=True)
            d = (x2 - mm) + e2
            cmin = jnp.min(d, axis=1, keepdims=True)
            iota = (lax.broadcasted_iota(jnp.int32, (N_T, E_C), 1)
                    + c * E_C).astype(jnp.float32)
            cidx = jnp.min(
                jnp.where(d == cmin, iota, jnp.inf),
                axis=1, keepdims=True)
            better = cmin < run_min
            run_min = jnp.where(better, cmin, run_min)
            run_idx = jnp.where(better, cidx, run_idx)
        halves.append((run_min, run_idx))
    (v0, i0), (v1, i1) = halves
    v0_carry = v0.astype(jnp.bfloat16).astype(jnp.float32)
    pick1 = v1 < v0_carry
    run_idx = jnp.where(pick1, i1, i0)
    run_min = jnp.where(pick1, v1, v0)
    ind_ref[...] = run_idx[:, 0].astype(jnp.int32)

    step = pl.program_id(0)

    @pl.when(step == 0)
    def _():
        diff_ref[0, 0] = 0.0

    diff_ref[0, 0] += jnp.sum(run_min)

    @pl.when(step == pl.num_programs(0) - 1)
    def _():
        diff_ref[0, 0] = diff_ref[0, 0] * (1.0 / (N_TOKENS * DIM))


def _argmin_call(flat_x, embed):
    grid = (N_TOKENS // N_T,)
    return pl.pallas_call(
        _argmin_body,
        grid=grid,
        in_specs=[
            pl.BlockSpec((N_T, DIM), lambda i: (i, 0)),
            pl.BlockSpec((DIM, N_EMBED), lambda i: (0, 0)),
        ],
        out_specs=[
            pl.BlockSpec((N_T,), lambda i: (i,)),
            pl.BlockSpec(memory_space=pltpu.SMEM),
        ],
        out_shape=[
            jax.ShapeDtypeStruct((N_TOKENS,), jnp.int32),
            jax.ShapeDtypeStruct((1, 1), jnp.float32),
        ],
    )(flat_x, embed)


def _gather_call(table, ind):
    info = plsc.get_sparse_core_info()
    nw = info.num_cores * info.num_subcores
    b_per_w = N_TOKENS // nw
    n_chunks = b_per_w // 128
    mesh = plsc.VectorSubcoreMesh(core_axis_name="c", subcore_axis_name="s")

    @functools.partial(
        pl.kernel, mesh=mesh,
        compiler_params=pltpu.CompilerParams(use_tc_tiling_on_sc=False),
        out_type=jax.ShapeDtypeStruct((N_TOKENS, DIM), jnp.float32),
        scratch_types=[
            pltpu.VMEM((b_per_w,), jnp.int32),
            pltpu.VMEM((b_per_w, DIM), jnp.float32),
            pltpu.SemaphoreType.DMA,
        ],
    )
    def k(table_hbm, idx_hbm, out_hbm, idx_v, rows_v, sem):
        wid = lax.axis_index("s") * info.num_cores + lax.axis_index("c")
        base = wid * b_per_w
        pltpu.sync_copy(idx_hbm.at[pl.ds(base, b_per_w)], idx_v)
        copies = [
            pltpu.async_copy(
                table_hbm.at[idx_v.at[pl.ds(g * 128, 128)]],
                rows_v.at[pl.ds(g * 128, 128)],
                sem,
            )
            for g in range(n_chunks)
        ]
        for c in copies:
            c.wait()
        pltpu.sync_copy(rows_v, out_hbm.at[pl.ds(base, b_per_w)])

    return k(table, ind)


def kernel(x, embed):
    flat_x = x.reshape(-1, DIM)
    ind, diff = _argmin_call(flat_x, embed)
    table = embed.T
    quantize = _gather_call(table, ind)
    return (
        quantize.reshape(x.shape),
        diff.reshape(()),
        ind.reshape(x.shape[:-1]),
    )


# speedup vs baseline: 1.3179x; 1.0128x over previous
import functools

import jax
import jax.numpy as jnp
from jax import lax
from jax.experimental import pallas as pl
from jax.experimental.pallas import tpu as pltpu
from jax.experimental.pallas import tpu_sc as plsc

DIM = 32
N_EMBED = 8192
N_TOKENS = 16384

N_T = 2048
E_C = 2048


def _argmin_body(x_ref, emb_ref, ind_ref, diff_ref):
    x = x_ref[...]
    x2 = jnp.sum(x * x, axis=1, keepdims=True)
    xb = (2.0 * x).astype(jnp.bfloat16)
    n_chunks_half = (N_EMBED // 2) // E_C
    halves = []
    for h in range(2):
        run_min = jnp.full((N_T, 1), jnp.inf, jnp.float32)
        run_idx = jnp.full((N_T, 1), jnp.inf, jnp.float32)
        for c in range(h * n_chunks_half, (h + 1) * n_chunks_half):
            e_blk = emb_ref[:, c * E_C:(c + 1) * E_C]
            mm = jnp.dot(xb, e_blk, preferred_element_type=jnp.float32)
            e2 = jnp.sum(e_blk * e_blk, axis=0, keepdims=True)
            d = (x2 - mm) + e2
            cmin = jnp.min(d, axis=1, keepdims=True)
            iota = (lax.broadcasted_iota(jnp.int32, (N_T, E_C), 1)
                    + c * E_C).astype(jnp.float32)
            cidx = jnp.min(
                jnp.where(d == cmin, iota, jnp.inf),
                axis=1, keepdims=True)
            better = cmin < run_min
            run_min = jnp.where(better, cmin, run_min)
            run_idx = jnp.where(better, cidx, run_idx)
        halves.append((run_min, run_idx))
    (v0, i0), (v1, i1) = halves
    v0_carry = v0.astype(jnp.bfloat16).astype(jnp.float32)
    pick1 = v1 < v0_carry
    run_idx = jnp.where(pick1, i1, i0)
    run_min = jnp.where(pick1, v1, v0)
    ind_ref[...] = run_idx[:, 0].astype(jnp.int32)

    step = pl.program_id(0)

    @pl.when(step == 0)
    def _():
        diff_ref[0, 0] = 0.0

    diff_ref[0, 0] += jnp.sum(run_min)

    @pl.when(step == pl.num_programs(0) - 1)
    def _():
        diff_ref[0, 0] = diff_ref[0, 0] * (1.0 / (N_TOKENS * DIM))


def _argmin_call(flat_x, embed):
    grid = (N_TOKENS // N_T,)
    return pl.pallas_call(
        _argmin_body,
        grid=grid,
        in_specs=[
            pl.BlockSpec((N_T, DIM), lambda i: (i, 0)),
            pl.BlockSpec((DIM, N_EMBED), lambda i: (0, 0)),
        ],
        out_specs=[
            pl.BlockSpec((N_T,), lambda i: (i,)),
            pl.BlockSpec(memory_space=pltpu.SMEM),
        ],
        out_shape=[
            jax.ShapeDtypeStruct((N_TOKENS,), jnp.int32),
            jax.ShapeDtypeStruct((1, 1), jnp.float32),
        ],
    )(flat_x, embed)


def _gather_call(table, ind):
    info = plsc.get_sparse_core_info()
    nw = info.num_cores * info.num_subcores
    b_per_w = N_TOKENS // nw
    n_chunks = b_per_w // 128
    mesh = plsc.VectorSubcoreMesh(core_axis_name="c", subcore_axis_name="s")

    @functools.partial(
        pl.kernel, mesh=mesh,
        compiler_params=pltpu.CompilerParams(use_tc_tiling_on_sc=False),
        out_type=jax.ShapeDtypeStruct((N_TOKENS, DIM), jnp.float32),
        scratch_types=[
            pltpu.VMEM((b_per_w,), jnp.int32),
            pltpu.VMEM((b_per_w, DIM), jnp.float32),
            pltpu.SemaphoreType.DMA,
        ],
    )
    def k(table_hbm, idx_hbm, out_hbm, idx_v, rows_v, sem):
        wid = lax.axis_index("s") * info.num_cores + lax.axis_index("c")
        base = wid * b_per_w
        pltpu.sync_copy(idx_hbm.at[pl.ds(base, b_per_w)], idx_v)
        copies = [
            pltpu.async_copy(
                table_hbm.at[idx_v.at[pl.ds(g * 128, 128)]],
                rows_v.at[pl.ds(g * 128, 128)],
                sem,
            )
            for g in range(n_chunks)
        ]
        for c in copies:
            c.wait()
        pltpu.sync_copy(rows_v, out_hbm.at[pl.ds(base, b_per_w)])

    return k(table, ind)


def kernel(x, embed):
    flat_x = x.reshape(-1, DIM)
    ind, diff = _argmin_call(flat_x, embed)
    table = embed.T
    quantize = _gather_call(table, ind)
    return (
        quantize.reshape(x.shape),
        diff.reshape(()),
        ind.reshape(x.shape[:-1]),
    )
